# baseline (speedup 1.0000x reference)
import functools

import jax
import jax.numpy as jnp
from jax import lax
from jax.experimental import pallas as pl
from jax.experimental.pallas import tpu as pltpu
from jax.experimental.pallas import tpu_sc as plsc

N_NODES = 100000
N_EDGES = 3200000

NW = 32
EPW = N_EDGES // NW
C = 10000
NCH = EPW // C
C1 = 2000
NCH1 = EPW // C1
EPW2 = 2 * EPW
C2 = 2000
NCH2 = EPW2 // C2

NPAD = 100352
NPT = NPAD // 16
TC_R = NPAD // 128
LANES = 128

_MESH = plsc.VectorSubcoreMesh(core_axis_name="c", subcore_axis_name="s")


def _stage_node_slices(sid, copies):
    off = sid * NPT
    for src, dst in copies:
        pltpu.sync_copy(src.at[pl.ds(off, NPT)], dst.at[pl.ds(off, NPT)])


def _local_gather(tab_loc, src_v, val_v, c):
    def gbody(j, carry):
        o = pl.multiple_of(j * 16, 16)
        idx = src_v[pl.ds(o, 16)]
        rows = lax.shift_right_logical(idx, 7)
        cols = lax.bitwise_and(idx, 127)
        val_v[pl.ds(o, 16)] = plsc.load_gather(tab_loc, [rows, cols])
        return carry

    lax.fori_loop(0, c // 16, gbody, 0)


@functools.partial(
    pl.kernel,
    out_type=jax.ShapeDtypeStruct((2 * NPAD,), jnp.float32),
    mesh=_MESH,
    scratch_types=[
        pltpu.VMEM((C,), jnp.int32),
        pltpu.VMEM((C,), jnp.int32),
        pltpu.VMEM((C,), jnp.float32),
        pltpu.VMEM_SHARED((NPAD,), jnp.float32),
        pltpu.SemaphoreType.DMA,
        pltpu.SemaphoreType.DMA,
    ],
)
def _deg_kernel(dst_hbm, ones_hbm, zeros_hbm, out_hbm,
                idx0_v, idx1_v, ones_v, acc_sh, sc0, sc1):
    cid = lax.axis_index("c")
    sid = lax.axis_index("s")
    w = sid * 2 + cid
    pltpu.sync_copy(ones_hbm, ones_v)
    _stage_node_slices(sid, [(zeros_hbm, acc_sh)])
    plsc.subcore_barrier()

    e0 = w * EPW
    bufs = ((idx0_v, sc0), (idx1_v, sc1))

    def body(i, carry):
        for b, (idx_v, sc) in enumerate(bufs):
            @pl.when(i > 0)
            def _():
                pltpu.make_async_copy(ones_v, acc_sh.at[idx_v], sc).wait()

            pltpu.sync_copy(dst_hbm.at[pl.ds(e0 + (2 * i + b) * C, C)], idx_v)
            pltpu.async_copy(ones_v, acc_sh.at[idx_v], sc, add=True)
        return carry

    lax.fori_loop(0, NCH // 2, body, 0)
    for idx_v, sc in bufs:
        pltpu.make_async_copy(ones_v, acc_sh.at[idx_v], sc).wait()

    plsc.subcore_barrier()
    off = sid * NPT
    pltpu.sync_copy(acc_sh.at[pl.ds(off, NPT)],
                    out_hbm.at[pl.ds(cid * NPAD + off, NPT)])


@functools.partial(
    pl.kernel,
    out_type=jax.ShapeDtypeStruct((2 * NPAD,), jnp.float32),
    mesh=_MESH,
    scratch_types=[
        pltpu.VMEM((TC_R, LANES), jnp.float32),
        pltpu.VMEM((C1,), jnp.int32),
        pltpu.VMEM((C1,), jnp.int32),
        pltpu.VMEM((C1,), jnp.int32),
        pltpu.VMEM((C1,), jnp.int32),
        pltpu.VMEM((C1,), jnp.float32),
        pltpu.VMEM((C1,), jnp.float32),
        pltpu.VMEM_SHARED((NPAD,), jnp.float32),
        pltpu.SemaphoreType.DMA,
        pltpu.SemaphoreType.DMA,
    ],
    compiler_params=pltpu.CompilerParams(needs_layout_passes=False),
)
def _prop1_kernel(src_hbm, dst_hbm, w_hbm, zeros_hbm, out_hbm,
                  tab_loc, src0_v, src1_v, dst0_v, dst1_v, val0_v, val1_v,
                  acc_sh, sc0, sc1):
    cid = lax.axis_index("c")
    sid = lax.axis_index("s")
    w = sid * 2 + cid
    pltpu.sync_copy(w_hbm, tab_loc)
    _stage_node_slices(sid, [(zeros_hbm, acc_sh)])
    plsc.subcore_barrier()

    e0 = w * EPW
    bufs = ((src0_v, dst0_v, val0_v, sc0), (src1_v, dst1_v, val1_v, sc1))

    def _chunk(k0, src_v, dst_v, val_v, sc):
        pltpu.sync_copy(src_hbm.at[pl.ds(k0, C1)], src_v)
        pltpu.sync_copy(dst_hbm.at[pl.ds(k0, C1)], dst_v)
        _local_gather(tab_loc, src_v, val_v, C1)
        pltpu.async_copy(val_v, acc_sh.at[dst_v], sc, add=True)

    def body(i, carry):
        for b, (src_v, dst_v, val_v, sc) in enumerate(bufs):
            @pl.when(i > 0)
            def _():
                pltpu.make_async_copy(val_v, acc_sh.at[dst_v], sc).wait()

            _chunk(e0 + (2 * i + b) * C1, src_v, dst_v, val_v, sc)
        return carry

    lax.fori_loop(0, NCH1 // 2, body, 0)
    for src_v, dst_v, val_v, sc in bufs:
        pltpu.make_async_copy(val_v, acc_sh.at[dst_v], sc).wait()

    plsc.subcore_barrier()
    off = sid * NPT
    pltpu.sync_copy(acc_sh.at[pl.ds(off, NPT)],
                    out_hbm.at[pl.ds(cid * NPAD + off, NPT)])


@functools.partial(
    pl.kernel,
    out_type=jax.ShapeDtypeStruct((4 * NPAD,), jnp.float32),
    mesh=_MESH,
    scratch_types=[
        pltpu.VMEM((TC_R, LANES), jnp.float32),
        pltpu.VMEM((C2,), jnp.int32),
        pltpu.VMEM((C2,), jnp.int32),
        pltpu.VMEM((C2,), jnp.int32),
        pltpu.VMEM((C2,), jnp.int32),
        pltpu.VMEM((C2,), jnp.float32),
        pltpu.VMEM((C2,), jnp.float32),
        pltpu.VMEM_SHARED((NPAD,), jnp.float32),
        pltpu.VMEM_SHARED((NPAD,), jnp.float32),
        pltpu.SemaphoreType.DMA,
        pltpu.SemaphoreType.DMA,
    ],
    compiler_params=pltpu.CompilerParams(needs_layout_passes=False),
)
def _prop2_kernel(src_hbm, dst_hbm, wa_hbm, wb_hbm, zeros_hbm, out_hbm,
                  tab_loc, src0_v, src1_v, dst0_v, dst1_v, val0_v, val1_v,
                  acca_sh, accb_sh, sc0, sc1):
    cid = lax.axis_index("c")
    sid = lax.axis_index("s")
    is_a = sid < 8

    @pl.when(is_a)
    def _():
        pltpu.sync_copy(wa_hbm, tab_loc)

    @pl.when(jnp.logical_not(is_a))
    def _():
        pltpu.sync_copy(wb_hbm, tab_loc)

    _stage_node_slices(sid, [(zeros_hbm, acca_sh), (zeros_hbm, accb_sh)])
    plsc.subcore_barrier()

    e0 = ((sid % 8) * 2 + cid) * EPW2
    bufs = ((src0_v, dst0_v, val0_v, sc0), (src1_v, dst1_v, val1_v, sc1))

    def _wait(dst_v, val_v, sc):
        @pl.when(is_a)
        def _():
            pltpu.make_async_copy(val_v, acca_sh.at[dst_v], sc).wait()

        @pl.when(jnp.logical_not(is_a))
        def _():
            pltpu.make_async_copy(val_v, accb_sh.at[dst_v], sc).wait()

    def body(i, carry):
        for b, (src_v, dst_v, val_v, sc) in enumerate(bufs):
            @pl.when(i > 0)
            def _():
                _wait(dst_v, val_v, sc)

            k0 = e0 + (2 * i + b) * C2
            pltpu.sync_copy(src_hbm.at[pl.ds(k0, C2)], src_v)
            pltpu.sync_copy(dst_hbm.at[pl.ds(k0, C2)], dst_v)
            _local_gather(tab_loc, src_v, val_v, C2)

            @pl.when(is_a)
            def _():
                pltpu.async_copy(val_v, acca_sh.at[dst_v], sc, add=True)

            @pl.when(jnp.logical_not(is_a))
            def _():
                pltpu.async_copy(val_v, accb_sh.at[dst_v], sc, add=True)

        return carry

    lax.fori_loop(0, NCH2 // 2, body, 0)
    for src_v, dst_v, val_v, sc in bufs:
        _wait(dst_v, val_v, sc)

    plsc.subcore_barrier()
    off = sid * NPT
    pltpu.sync_copy(acca_sh.at[pl.ds(off, NPT)],
                    out_hbm.at[pl.ds(cid * NPAD + off, NPT)])
    pltpu.sync_copy(accb_sh.at[pl.ds(off, NPT)],
                    out_hbm.at[pl.ds((2 + cid) * NPAD + off, NPT)])


def _tc_prep_body(deg_ref, x_ref, dinv_ref, w_ref):
    d = deg_ref[0] + deg_ref[1] + 1.0
    dinv = lax.rsqrt(d)
    dinv_ref[...] = dinv
    w_ref[...] = dinv * x_ref[...]


def _tc_layer1_body(g1_ref, w_ref, dinv_ref, w1_ref, b1_ref, wa_ref, wb_ref):
    dinv = dinv_ref[...]
    p1 = dinv * (g1_ref[0] + g1_ref[1] + w_ref[...])
    ha = jnp.maximum(p1 * w1_ref[0, 0] + b1_ref[0, 0], 0.0)
    hb = jnp.maximum(p1 * w1_ref[0, 1] + b1_ref[0, 1], 0.0)
    wa_ref[...] = dinv * ha
    wb_ref[...] = dinv * hb


def _tc_final_body(g2_ref, wa_ref, wb_ref, dinv_ref, w2_ref, b2_ref,
                   oa_ref, ob_ref):
    dinv = dinv_ref[...]
    ua = dinv * (g2_ref[0] + g2_ref[1] + wa_ref[...])
    ub = dinv * (g2_ref[2] + g2_ref[3] + wb_ref[...])
    oa_ref[...] = ua * w2_ref[0, 0] + ub * w2_ref[1, 0] + b2_ref[0, 0]
    ob_ref[...] = ua * w2_ref[0, 1] + ub * w2_ref[1, 1] + b2_ref[0, 1]


def _vspec():
    return pl.BlockSpec(memory_space=pltpu.VMEM)


def _sspec():
    return pl.BlockSpec(memory_space=pltpu.SMEM)


_f32 = jnp.float32


def kernel(x, edge_index, W1, b1, W2, b2):
    n = x.shape[0]
    assert n == N_NODES and edge_index.shape[1] == N_EDGES
    src = edge_index[0].astype(jnp.int32)
    dst = edge_index[1].astype(jnp.int32)
    xp = jnp.pad(x[:, 0], (0, NPAD - n))
    zeros = jnp.zeros((NPAD,), _f32)
    ones = jnp.ones((C,), _f32)

    degp = _deg_kernel(dst, ones, zeros)

    dinv, w1v = pl.pallas_call(
        _tc_prep_body,
        out_shape=[jax.ShapeDtypeStruct((TC_R, LANES), _f32)] * 2,
        in_specs=[_vspec(), _vspec()],
        out_specs=[_vspec(), _vspec()],
    )(degp.reshape(2, TC_R, LANES), xp.reshape(TC_R, LANES))

    g1p = _prop1_kernel(src, dst, w1v, zeros)

    w2a, w2b = pl.pallas_call(
        _tc_layer1_body,
        out_shape=[jax.ShapeDtypeStruct((TC_R, LANES), _f32)] * 2,
        in_specs=[_vspec(), _vspec(), _vspec(), _sspec(), _sspec()],
        out_specs=[_vspec(), _vspec()],
    )(g1p.reshape(2, TC_R, LANES), w1v, dinv,
      W1.reshape(1, 2), b1.reshape(1, 2))

    g2p = _prop2_kernel(src, dst, w2a, w2b, zeros)

    oa, ob = pl.pallas_call(
        _tc_final_body,
        out_shape=[jax.ShapeDtypeStruct((TC_R, LANES), _f32)] * 2,
        in_specs=[_vspec()] * 4 + [_sspec(), _sspec()],
        out_specs=[_vspec(), _vspec()],
    )(g2p.reshape(4, TC_R, LANES), w2a, w2b, dinv,
      W2.reshape(2, 2), b2.reshape(1, 2))

    return jnp.stack([oa.reshape(NPAD)[:n], ob.reshape(NPAD)[:n]], axis=-1)


# speedup vs baseline: 1.3254x; 1.3254x over previous
import functools

import jax
import jax.numpy as jnp
from jax import lax
from jax.experimental import pallas as pl
from jax.experimental.pallas import tpu as pltpu
from jax.experimental.pallas import tpu_sc as plsc

N_NODES = 100000
N_EDGES = 3200000

NW = 32
EPW = N_EDGES // NW
C1 = 10000
NCH1 = EPW // C1
C2 = 5000
NCH2 = EPW // C2

NPAD = 100352
NPT = NPAD // 16
TC_R = NPAD // 128
LANES = 128

_MESH = plsc.VectorSubcoreMesh(core_axis_name="c", subcore_axis_name="s")
_f32 = jnp.float32


def _rsqrt16(d):
    magic = jnp.full((16,), 0x5F3759DF, jnp.int32)
    half = jnp.full((16,), 0.5, _f32)
    three_half = jnp.full((16,), 1.5, _f32)
    y = plsc.bitcast(magic - lax.shift_right_logical(plsc.bitcast(d, jnp.int32), 1),
                     _f32)
    hd = half * d
    for _ in range(3):
        y = y * (three_half - hd * y * y)
    return y


def _vec_loop(n, body):
    def wrapped(j, carry):
        body(pl.multiple_of(j * 16, 16))
        return carry

    lax.fori_loop(0, n // 16, wrapped, 0)


@functools.partial(
    pl.kernel,
    out_type=jax.ShapeDtypeStruct((2 * NPAD,), _f32),
    mesh=_MESH,
    scratch_types=[
        pltpu.VMEM((C1,), jnp.int32),
        pltpu.VMEM((C1,), jnp.int32),
        pltpu.VMEM((C1,), _f32),
        pltpu.VMEM_SHARED((NPAD,), _f32),
        pltpu.SemaphoreType.DMA,
        pltpu.SemaphoreType.DMA,
    ],
)
def _deg_kernel(dst_hbm, ones_hbm, zeros_hbm, out_hbm,
                idx0_v, idx1_v, ones_v, acc_sh, sc0, sc1):
    cid = lax.axis_index("c")
    sid = lax.axis_index("s")
    w = sid * 2 + cid
    off = sid * NPT
    pltpu.sync_copy(ones_hbm, ones_v)
    pltpu.sync_copy(zeros_hbm.at[pl.ds(off, NPT)], acc_sh.at[pl.ds(off, NPT)])
    plsc.subcore_barrier()

    e0 = w * EPW
    bufs = ((idx0_v, sc0), (idx1_v, sc1))

    def body(i, carry):
        for b, (idx_v, sc) in enumerate(bufs):
            @pl.when(i > 0)
            def _():
                pltpu.make_async_copy(ones_v, acc_sh.at[idx_v], sc).wait()

            pltpu.sync_copy(dst_hbm.at[pl.ds(e0 + (2 * i + b) * C1, C1)], idx_v)
            pltpu.async_copy(ones_v, acc_sh.at[idx_v], sc, add=True)
        return carry

    lax.fori_loop(0, NCH1 // 2, body, 0)
    for idx_v, sc in bufs:
        pltpu.make_async_copy(ones_v, acc_sh.at[idx_v], sc).wait()

    plsc.subcore_barrier()
    pltpu.sync_copy(acc_sh.at[pl.ds(off, NPT)],
                    out_hbm.at[pl.ds(cid * NPAD + off, NPT)])


@functools.partial(
    pl.kernel,
    out_type=jax.ShapeDtypeStruct((2 * NPAD,), _f32),
    mesh=_MESH,
    scratch_types=[
        pltpu.VMEM((C1,), jnp.int32),
        pltpu.VMEM((C1,), jnp.int32),
        pltpu.VMEM((C1,), jnp.int32),
        pltpu.VMEM((C1,), jnp.int32),
        pltpu.VMEM((C1,), _f32),
        pltpu.VMEM((C1,), _f32),
        pltpu.VMEM((NPT,), _f32),
        pltpu.VMEM((NPT,), _f32),
        pltpu.VMEM((NPT,), _f32),
        pltpu.VMEM((NPT,), _f32),
        pltpu.VMEM_SHARED((NPAD,), _f32),
        pltpu.VMEM_SHARED((NPAD,), _f32),
        pltpu.SemaphoreType.DMA,
        pltpu.SemaphoreType.DMA,
    ],
    compiler_params=pltpu.CompilerParams(needs_layout_passes=False),
)
def _prop1_kernel(src_hbm, dst_hbm, degp_hbm, xp_hbm, zeros_hbm, out_hbm,
                  src0_v, src1_v, dst0_v, dst1_v, val0_v, val1_v,
                  d0_v, d1_v, x_v, w_v,
                  tab_sh, acc_sh, sc0, sc1):
    cid = lax.axis_index("c")
    sid = lax.axis_index("s")
    w = sid * 2 + cid
    off = sid * NPT
    pltpu.sync_copy(degp_hbm.at[pl.ds(off, NPT)], d0_v)
    pltpu.sync_copy(degp_hbm.at[pl.ds(NPAD + off, NPT)], d1_v)
    pltpu.sync_copy(xp_hbm.at[pl.ds(off, NPT)], x_v)
    pltpu.sync_copy(zeros_hbm.at[pl.ds(off, NPT)], acc_sh.at[pl.ds(off, NPT)])
    one = jnp.full((16,), 1.0, _f32)

    def welt(o):
        d = d0_v[pl.ds(o, 16)] + d1_v[pl.ds(o, 16)] + one
        w_v[pl.ds(o, 16)] = _rsqrt16(d) * x_v[pl.ds(o, 16)]

    _vec_loop(NPT, welt)
    pltpu.sync_copy(w_v, tab_sh.at[pl.ds(off, NPT)])
    plsc.subcore_barrier()

    e0 = w * EPW
    bufs = ((src0_v, dst0_v, val0_v, sc0), (src1_v, dst1_v, val1_v, sc1))

    def body(i, carry):
        for b, (src_v, dst_v, val_v, sc) in enumerate(bufs):
            @pl.when(i > 0)
            def _():
                pltpu.make_async_copy(val_v, acc_sh.at[dst_v], sc).wait()

            k0 = e0 + (2 * i + b) * C1
            pltpu.sync_copy(src_hbm.at[pl.ds(k0, C1)], src_v)
            pltpu.sync_copy(dst_hbm.at[pl.ds(k0, C1)], dst_v)
            pltpu.sync_copy(tab_sh.at[src_v], val_v)
            pltpu.async_copy(val_v, acc_sh.at[dst_v], sc, add=True)
        return carry

    lax.fori_loop(0, NCH1 // 2, body, 0)
    for src_v, dst_v, val_v, sc in bufs:
        pltpu.make_async_copy(val_v, acc_sh.at[dst_v], sc).wait()

    plsc.subcore_barrier()
    pltpu.sync_copy(acc_sh.at[pl.ds(off, NPT)],
                    out_hbm.at[pl.ds(cid * NPAD + off, NPT)])


@functools.partial(
    pl.kernel,
    out_type=[jax.ShapeDtypeStruct((4 * NPAD,), _f32),
              jax.ShapeDtypeStruct((2 * NPAD,), _f32)],
    mesh=_MESH,
    scratch_types=[
        pltpu.VMEM((C2,), jnp.int32),
        pltpu.VMEM((C2,), jnp.int32),
        pltpu.VMEM((C2,), jnp.int32),
        pltpu.VMEM((C2,), jnp.int32),
        pltpu.VMEM((C2,), _f32),
        pltpu.VMEM((C2,), _f32),
        pltpu.VMEM((C2,), _f32),
        pltpu.VMEM((C2,), _f32),
        pltpu.VMEM((NPT,), _f32),
        pltpu.VMEM((NPT,), _f32),
        pltpu.VMEM((NPT,), _f32),
        pltpu.VMEM((NPT,), _f32),
        pltpu.VMEM((NPT,), _f32),
        pltpu.VMEM((NPT,), _f32),
        pltpu.VMEM((64,), _f32),
        pltpu.VMEM_SHARED((NPAD,), _f32),
        pltpu.VMEM_SHARED((NPAD,), _f32),
        pltpu.VMEM_SHARED((NPAD,), _f32),
        pltpu.VMEM_SHARED((NPAD,), _f32),
        pltpu.SemaphoreType.DMA,
        pltpu.SemaphoreType.DMA,
        pltpu.SemaphoreType.DMA,
        pltpu.SemaphoreType.DMA,
    ],
    compiler_params=pltpu.CompilerParams(needs_layout_passes=False),
)
def _prop2_kernel(src_hbm, dst_hbm, degp_hbm, xp_hbm, g1p_hbm, par_hbm,
                  zeros_hbm, out_hbm, outw_hbm,
                  src0_v, src1_v, dst0_v, dst1_v,
                  va0_v, va1_v, vb0_v, vb1_v,
                  d0_v, d1_v, x_v, g0_v, wa_v, wb_v, par_v,
                  taba_sh, tabb_sh, acca_sh, accb_sh,
                  sa0, sa1, sb0, sb1):
    cid = lax.axis_index("c")
    sid = lax.axis_index("s")
    w = sid * 2 + cid
    off = sid * NPT
    pltpu.sync_copy(degp_hbm.at[pl.ds(off, NPT)], d0_v)
    pltpu.sync_copy(degp_hbm.at[pl.ds(NPAD + off, NPT)], d1_v)
    pltpu.sync_copy(xp_hbm.at[pl.ds(off, NPT)], x_v)
    pltpu.sync_copy(par_hbm, par_v)
    pltpu.sync_copy(zeros_hbm.at[pl.ds(off, NPT)], acca_sh.at[pl.ds(off, NPT)])
    pltpu.sync_copy(zeros_hbm.at[pl.ds(off, NPT)], accb_sh.at[pl.ds(off, NPT)])
    one = jnp.full((16,), 1.0, _f32)
    zero = jnp.full((16,), 0.0, _f32)
    w1a = par_v[pl.ds(0, 16)]
    w1b = par_v[pl.ds(16, 16)]
    b1a = par_v[pl.ds(32, 16)]
    b1b = par_v[pl.ds(48, 16)]
    pltpu.sync_copy(g1p_hbm.at[pl.ds(off, NPT)], g0_v)
    pltpu.sync_copy(g1p_hbm.at[pl.ds(NPAD + off, NPT)], wa_v)

    def welt(o):
        d = d0_v[pl.ds(o, 16)] + d1_v[pl.ds(o, 16)] + one
        dinv = _rsqrt16(d)
        wloc = dinv * x_v[pl.ds(o, 16)]
        p1 = dinv * (g0_v[pl.ds(o, 16)] + wa_v[pl.ds(o, 16)] + wloc)
        ha = jnp.maximum(p1 * w1a + b1a, zero)
        hb = jnp.maximum(p1 * w1b + b1b, zero)
        wb_v[pl.ds(o, 16)] = dinv * hb
        x_v[pl.ds(o, 16)] = dinv * ha

    _vec_loop(NPT, welt)
    pltpu.sync_copy(x_v, taba_sh.at[pl.ds(off, NPT)])
    pltpu.sync_copy(wb_v, tabb_sh.at[pl.ds(off, NPT)])

    @pl.when(cid == 0)
    def _():
        pltpu.sync_copy(x_v, outw_hbm.at[pl.ds(off, NPT)])
        pltpu.sync_copy(wb_v, outw_hbm.at[pl.ds(NPAD + off, NPT)])

    plsc.subcore_barrier()

    e0 = w * EPW
    bufs = ((src0_v, dst0_v, va0_v, vb0_v, sa0, sb0),
            (src1_v, dst1_v, va1_v, vb1_v, sa1, sb1))

    def body(i, carry):
        for b, (src_v, dst_v, va_v, vb_v, sa, sb) in enumerate(bufs):
            @pl.when(i > 0)
            def _():
                pltpu.make_async_copy(va_v, acca_sh.at[dst_v], sa).wait()
                pltpu.make_async_copy(vb_v, accb_sh.at[dst_v], sb).wait()

            k0 = e0 + (2 * i + b) * C2
            pltpu.sync_copy(src_hbm.at[pl.ds(k0, C2)], src_v)
            pltpu.sync_copy(dst_hbm.at[pl.ds(k0, C2)], dst_v)
            pltpu.sync_copy(taba_sh.at[src_v], va_v)
            pltpu.sync_copy(tabb_sh.at[src_v], vb_v)
            pltpu.async_copy(va_v, acca_sh.at[dst_v], sa, add=True)
            pltpu.async_copy(vb_v, accb_sh.at[dst_v], sb, add=True)
        return carry

    lax.fori_loop(0, NCH2 // 2, body, 0)
    for src_v, dst_v, va_v, vb_v, sa, sb in bufs:
        pltpu.make_async_copy(va_v, acca_sh.at[dst_v], sa).wait()
        pltpu.make_async_copy(vb_v, accb_sh.at[dst_v], sb).wait()

    plsc.subcore_barrier()
    pltpu.sync_copy(acca_sh.at[pl.ds(off, NPT)],
                    out_hbm.at[pl.ds(cid * NPAD + off, NPT)])
    pltpu.sync_copy(accb_sh.at[pl.ds(off, NPT)],
                    out_hbm.at[pl.ds((2 + cid) * NPAD + off, NPT)])


def _tc_final_body(g2_ref, wab_ref, deg_ref, w2_ref, b2_ref, oa_ref, ob_ref):
    dinv = lax.rsqrt(deg_ref[0] + deg_ref[1] + 1.0)
    ua = dinv * (g2_ref[0] + g2_ref[1] + wab_ref[0])
    ub = dinv * (g2_ref[2] + g2_ref[3] + wab_ref[1])
    oa_ref[...] = ua * w2_ref[0, 0] + ub * w2_ref[1, 0] + b2_ref[0, 0]
    ob_ref[...] = ua * w2_ref[0, 1] + ub * w2_ref[1, 1] + b2_ref[0, 1]


def _vspec():
    return pl.BlockSpec(memory_space=pltpu.VMEM)


def _sspec():
    return pl.BlockSpec(memory_space=pltpu.SMEM)


def kernel(x, edge_index, W1, b1, W2, b2):
    n = x.shape[0]
    assert n == N_NODES and edge_index.shape[1] == N_EDGES
    src = edge_index[0].astype(jnp.int32)
    dst = edge_index[1].astype(jnp.int32)
    xp = jnp.pad(x[:, 0], (0, NPAD - n))
    zeros = jnp.zeros((NPAD,), _f32)
    ones = jnp.ones((C1,), _f32)
    par = jnp.concatenate([
        jnp.broadcast_to(W1[0, 0], (16,)),
        jnp.broadcast_to(W1[0, 1], (16,)),
        jnp.broadcast_to(b1[0], (16,)),
        jnp.broadcast_to(b1[1], (16,)),
    ]).astype(_f32)

    degp = _deg_kernel(dst, ones, zeros)
    g1p = _prop1_kernel(src, dst, degp, xp, zeros)
    g2p, wab = _prop2_kernel(src, dst, degp, xp, g1p, par, zeros)

    oa, ob = pl.pallas_call(
        _tc_final_body,
        out_shape=[jax.ShapeDtypeStruct((TC_R, LANES), _f32)] * 2,
        in_specs=[_vspec(), _vspec(), _vspec(), _sspec(), _sspec()],
        out_specs=[_vspec(), _vspec()],
    )(g2p.reshape(4, TC_R, LANES), wab.reshape(2, TC_R, LANES),
      degp.reshape(2, TC_R, LANES), W2.reshape(2, 2), b2.reshape(1, 2))

    return jnp.stack([oa.reshape(NPAD)[:n], ob.reshape(NPAD)[:n]], axis=-1)


# speedup vs baseline: 1.4204x; 1.0717x over previous
import functools

import jax
import jax.numpy as jnp
from jax import lax
from jax.experimental import pallas as pl
from jax.experimental.pallas import tpu as pltpu
from jax.experimental.pallas import tpu_sc as plsc

N_NODES = 100000
N_EDGES = 3200000

NW = 32
EPW = N_EDGES // NW
C0 = 25000
NCH0 = EPW // C0
C = 10000
NCH = EPW // C

NPAD = 100352
NPT = NPAD // 16
TC_R = NPAD // 128
LANES = 128

_MESH = plsc.VectorSubcoreMesh(core_axis_name="c", subcore_axis_name="s")
_f32 = jnp.float32


@functools.partial(
    pl.kernel,
    out_type=jax.ShapeDtypeStruct((2 * NPAD,), _f32),
    mesh=_MESH,
    scratch_types=[
        pltpu.VMEM((C0,), jnp.int32),
        pltpu.VMEM((C0,), jnp.int32),
        pltpu.VMEM((C0,), jnp.int32),
        pltpu.VMEM((C0,), _f32),
        pltpu.SemaphoreType.DMA,
        pltpu.SemaphoreType.DMA,
        pltpu.SemaphoreType.DMA,
        pltpu.SemaphoreType.DMA,
        pltpu.SemaphoreType.DMA,
        pltpu.VMEM_SHARED((NPAD,), _f32),
    ],
)
def _deg_kernel(dst_hbm, ones_hbm, zeros_hbm, out_hbm,
                idx0, idx1, idx2, ones_v, si0, si1, si2, ss0, ss1, acc_sh):
    cid = lax.axis_index("c")
    sid = lax.axis_index("s")
    w = sid * 2 + cid
    off = sid * NPT
    e0 = w * EPW
    idxs = (idx0, idx1, idx2)
    isems = (si0, si1, si2)
    ssems = (ss0, ss1)

    def _iload(k):
        pltpu.async_copy(dst_hbm.at[pl.ds(e0 + k * C0, C0)], idxs[k % 3],
                         isems[k % 3])

    def _iload_wait(k):
        pltpu.make_async_copy(dst_hbm.at[pl.ds(e0 + k * C0, C0)], idxs[k % 3],
                              isems[k % 3]).wait()

    def _scat(k):
        pltpu.async_copy(ones_v, acc_sh.at[idxs[k % 3]], ssems[k % 2], add=True)

    def _scat_wait(k):
        pltpu.make_async_copy(ones_v, acc_sh.at[idxs[k % 3]],
                              ssems[k % 2]).wait()

    for k in range(min(3, NCH0)):
        _iload(k)
    pltpu.sync_copy(ones_hbm, ones_v)
    pltpu.sync_copy(zeros_hbm.at[pl.ds(off, NPT)], acc_sh.at[pl.ds(off, NPT)])
    plsc.subcore_barrier()

    for k in range(NCH0):
        if k >= 2:
            _scat_wait(k - 2)
        if 3 <= k + 1 < NCH0:
            _iload(k + 1)
        _iload_wait(k)
        _scat(k)
    for k in range(max(0, NCH0 - 2), NCH0):
        _scat_wait(k)

    plsc.subcore_barrier()
    pltpu.sync_copy(acc_sh.at[pl.ds(off, NPT)],
                    out_hbm.at[pl.ds(cid * NPAD + off, NPT)])


@functools.partial(
    pl.kernel,
    out_type=jax.ShapeDtypeStruct((2 * NPAD,), _f32),
    mesh=_MESH,
    scratch_types=[
        pltpu.VMEM((C,), jnp.int32),
        pltpu.VMEM((C,), jnp.int32),
        pltpu.VMEM((C,), jnp.int32),
        pltpu.VMEM((C,), jnp.int32),
        pltpu.VMEM((C,), jnp.int32),
        pltpu.VMEM((C,), jnp.int32),
        pltpu.VMEM((C,), _f32),
        pltpu.VMEM((C,), _f32),
        pltpu.SemaphoreType.DMA,
        pltpu.SemaphoreType.DMA,
        pltpu.SemaphoreType.DMA,
        pltpu.SemaphoreType.DMA,
        pltpu.SemaphoreType.DMA,
        pltpu.SemaphoreType.DMA,
        pltpu.SemaphoreType.DMA,
        pltpu.SemaphoreType.DMA,
        pltpu.SemaphoreType.DMA,
        pltpu.VMEM_SHARED((NPAD,), _f32),
        pltpu.VMEM_SHARED((NPAD,), _f32),
    ],
)
def _prop1_kernel(src_hbm, dst_hbm, w_hbm, zeros_hbm, out_hbm,
                  srcb0, srcb1, srcb2, dstb0, dstb1, dstb2, valb0, valb1,
                  ssr0, ssr1, ssr2, sds0, sds1, sds2, sg, ss0, ss1,
                  tab_sh, acc_sh):
    cid = lax.axis_index("c")
    sid = lax.axis_index("s")
    w = sid * 2 + cid
    off = sid * NPT
    e0 = w * EPW
    srcs = (srcb0, srcb1, srcb2)
    dsts = (dstb0, dstb1, dstb2)
    vals = (valb0, valb1)
    srcsems = (ssr0, ssr1, ssr2)
    dstsems = (sds0, sds1, sds2)
    ssems = (ss0, ss1)

    def _iload(k):
        j = k % 3
        pltpu.async_copy(src_hbm.at[pl.ds(e0 + k * C, C)], srcs[j], srcsems[j])
        pltpu.async_copy(dst_hbm.at[pl.ds(e0 + k * C, C)], dsts[j], dstsems[j])

    def _iload_wait(k):
        j = k % 3
        pltpu.make_async_copy(src_hbm.at[pl.ds(e0 + k * C, C)], srcs[j],
                              srcsems[j]).wait()
        pltpu.make_async_copy(dst_hbm.at[pl.ds(e0 + k * C, C)], dsts[j],
                              dstsems[j]).wait()

    def _scat(k):
        pltpu.async_copy(vals[k % 2], acc_sh.at[dsts[k % 3]], ssems[k % 2],
                         add=True)

    def _scat_wait(k):
        pltpu.make_async_copy(vals[k % 2], acc_sh.at[dsts[k % 3]],
                              ssems[k % 2]).wait()

    for k in range(min(3, NCH)):
        _iload(k)
    pltpu.sync_copy(w_hbm.at[pl.ds(off, NPT)], tab_sh.at[pl.ds(off, NPT)])
    pltpu.sync_copy(zeros_hbm.at[pl.ds(off, NPT)], acc_sh.at[pl.ds(off, NPT)])
    plsc.subcore_barrier()

    for k in range(NCH):
        if k >= 2:
            _scat_wait(k - 2)
        if 3 <= k + 1 < NCH:
            _iload(k + 1)
        _iload_wait(k)
        pltpu.async_copy(tab_sh.at[srcs[k % 3]], vals[k % 2], sg).wait()
        _scat(k)
    for k in range(max(0, NCH - 2), NCH):
        _scat_wait(k)

    plsc.subcore_barrier()
    pltpu.sync_copy(acc_sh.at[pl.ds(off, NPT)],
                    out_hbm.at[pl.ds(cid * NPAD + off, NPT)])


@functools.partial(
    pl.kernel,
    out_type=jax.ShapeDtypeStruct((4 * NPAD,), _f32),
    mesh=_MESH,
    scratch_types=[
        pltpu.VMEM((C,), jnp.int32),
        pltpu.VMEM((C,), jnp.int32),
        pltpu.VMEM((C,), jnp.int32),
        pltpu.VMEM((C,), jnp.int32),
        pltpu.VMEM((C,), jnp.int32),
        pltpu.VMEM((C,), jnp.int32),
        pltpu.VMEM((C,), _f32),
        pltpu.VMEM((C,), _f32),
        pltpu.VMEM((C,), _f32),
        pltpu.VMEM((C,), _f32),
        pltpu.SemaphoreType.DMA,
        pltpu.SemaphoreType.DMA,
        pltpu.SemaphoreType.DMA,
        pltpu.SemaphoreType.DMA,
        pltpu.SemaphoreType.DMA,
        pltpu.SemaphoreType.DMA,
        pltpu.SemaphoreType.DMA,
        pltpu.SemaphoreType.DMA,
        pltpu.SemaphoreType.DMA,
        pltpu.SemaphoreType.DMA,
        pltpu.SemaphoreType.DMA,
        pltpu.SemaphoreType.DMA,
        pltpu.VMEM_SHARED((NPAD,), _f32),
        pltpu.VMEM_SHARED((NPAD,), _f32),
        pltpu.VMEM_SHARED((NPAD,), _f32),
        pltpu.VMEM_SHARED((NPAD,), _f32),
    ],
)
def _prop2_kernel(src_hbm, dst_hbm, wa_hbm, wb_hbm, zeros_hbm, out_hbm,
                  srcb0, srcb1, srcb2, dstb0, dstb1, dstb2,
                  vab0, vab1, vbb0, vbb1,
                  ssr0, ssr1, ssr2, sds0, sds1, sds2, sga, sgb,
                  ssa0, ssa1, ssb0, ssb1,
                  taba_sh, tabb_sh, acca_sh, accb_sh):
    cid = lax.axis_index("c")
    sid = lax.axis_index("s")
    w = sid * 2 + cid
    off = sid * NPT
    e0 = w * EPW
    srcs = (srcb0, srcb1, srcb2)
    dsts = (dstb0, dstb1, dstb2)
    vas = (vab0, vab1)
    vbs = (vbb0, vbb1)
    srcsems = (ssr0, ssr1, ssr2)
    dstsems = (sds0, sds1, sds2)
    sasems = (ssa0, ssa1)
    sbsems = (ssb0, ssb1)

    def _iload(k):
        j = k % 3
        pltpu.async_copy(src_hbm.at[pl.ds(e0 + k * C, C)], srcs[j], srcsems[j])
        pltpu.async_copy(dst_hbm.at[pl.ds(e0 + k * C, C)], dsts[j], dstsems[j])

    def _iload_wait(k):
        j = k % 3
        pltpu.make_async_copy(src_hbm.at[pl.ds(e0 + k * C, C)], srcs[j],
                              srcsems[j]).wait()
        pltpu.make_async_copy(dst_hbm.at[pl.ds(e0 + k * C, C)], dsts[j],
                              dstsems[j]).wait()

    def _scat(k):
        pltpu.async_copy(vas[k % 2], acca_sh.at[dsts[k % 3]], sasems[k % 2],
                         add=True)
        pltpu.async_copy(vbs[k % 2], accb_sh.at[dsts[k % 3]], sbsems[k % 2],
                         add=True)

    def _scat_wait(k):
        pltpu.make_async_copy(vas[k % 2], acca_sh.at[dsts[k % 3]],
                              sasems[k % 2]).wait()
        pltpu.make_async_copy(vbs[k % 2], accb_sh.at[dsts[k % 3]],
                              sbsems[k % 2]).wait()

    for k in range(min(3, NCH)):
        _iload(k)
    pltpu.sync_copy(wa_hbm.at[pl.ds(off, NPT)], taba_sh.at[pl.ds(off, NPT)])
    pltpu.sync_copy(wb_hbm.at[pl.ds(off, NPT)], tabb_sh.at[pl.ds(off, NPT)])
    pltpu.sync_copy(zeros_hbm.at[pl.ds(off, NPT)], acca_sh.at[pl.ds(off, NPT)])
    pltpu.sync_copy(zeros_hbm.at[pl.ds(off, NPT)], accb_sh.at[pl.ds(off, NPT)])
    plsc.subcore_barrier()

    for k in range(NCH):
        if k >= 2:
            _scat_wait(k - 2)
        if 3 <= k + 1 < NCH:
            _iload(k + 1)
        _iload_wait(k)
        ga = pltpu.make_async_copy(taba_sh.at[srcs[k % 3]], vas[k % 2], sga)
        ga.start()
        gb = pltpu.make_async_copy(tabb_sh.at[srcs[k % 3]], vbs[k % 2], sgb)
        gb.start()
        ga.wait()
        gb.wait()
        _scat(k)
    for k in range(max(0, NCH - 2), NCH):
        _scat_wait(k)

    plsc.subcore_barrier()
    pltpu.sync_copy(acca_sh.at[pl.ds(off, NPT)],
                    out_hbm.at[pl.ds(cid * NPAD + off, NPT)])
    pltpu.sync_copy(accb_sh.at[pl.ds(off, NPT)],
                    out_hbm.at[pl.ds((2 + cid) * NPAD + off, NPT)])


def _tc_prep_body(deg_ref, x_ref, dinv_ref, w_ref):
    d = deg_ref[0] + deg_ref[1] + 1.0
    dinv = lax.rsqrt(d)
    dinv_ref[...] = dinv
    w_ref[...] = dinv * x_ref[...]


def _tc_layer1_body(g1_ref, w_ref, dinv_ref, w1_ref, b1_ref, wa_ref, wb_ref):
    dinv = dinv_ref[...]
    p1 = dinv * (g1_ref[0] + g1_ref[1] + w_ref[...])
    ha = jnp.maximum(p1 * w1_ref[0, 0] + b1_ref[0, 0], 0.0)
    hb = jnp.maximum(p1 * w1_ref[0, 1] + b1_ref[0, 1], 0.0)
    wa_ref[...] = dinv * ha
    wb_ref[...] = dinv * hb


def _tc_final_body(g2_ref, wa_ref, wb_ref, dinv_ref, w2_ref, b2_ref,
                   oa_ref, ob_ref):
    dinv = dinv_ref[...]
    ua = dinv * (g2_ref[0] + g2_ref[1] + wa_ref[...])
    ub = dinv * (g2_ref[2] + g2_ref[3] + wb_ref[...])
    oa_ref[...] = ua * w2_ref[0, 0] + ub * w2_ref[1, 0] + b2_ref[0, 0]
    ob_ref[...] = ua * w2_ref[0, 1] + ub * w2_ref[1, 1] + b2_ref[0, 1]


def _vspec():
    return pl.BlockSpec(memory_space=pltpu.VMEM)


def _sspec():
    return pl.BlockSpec(memory_space=pltpu.SMEM)


def kernel(x, edge_index, W1, b1, W2, b2):
    n = x.shape[0]
    assert n == N_NODES and edge_index.shape[1] == N_EDGES
    src = edge_index[0].astype(jnp.int32)
    dst = edge_index[1].astype(jnp.int32)
    xp = jnp.pad(x[:, 0], (0, NPAD - n))
    zeros = jnp.zeros((NPAD,), _f32)
    ones = jnp.ones((C0,), _f32)

    degp = _deg_kernel(dst, ones, zeros)

    dinv, w1v = pl.pallas_call(
        _tc_prep_body,
        out_shape=[jax.ShapeDtypeStruct((TC_R, LANES), _f32)] * 2,
        in_specs=[_vspec(), _vspec()],
        out_specs=[_vspec(), _vspec()],
    )(degp.reshape(2, TC_R, LANES), xp.reshape(TC_R, LANES))

    g1p = _prop1_kernel(src, dst, w1v.reshape(NPAD), zeros)

    w2a, w2b = pl.pallas_call(
        _tc_layer1_body,
        out_shape=[jax.ShapeDtypeStruct((TC_R, LANES), _f32)] * 2,
        in_specs=[_vspec(), _vspec(), _vspec(), _sspec(), _sspec()],
        out_specs=[_vspec(), _vspec()],
    )(g1p.reshape(2, TC_R, LANES), w1v, dinv,
      W1.reshape(1, 2), b1.reshape(1, 2))

    g2p = _prop2_kernel(src, dst, w2a.reshape(NPAD), w2b.reshape(NPAD), zeros)

    oa, ob = pl.pallas_call(
        _tc_final_body,
        out_shape=[jax.ShapeDtypeStruct((TC_R, LANES), _f32)] * 2,
        in_specs=[_vspec()] * 4 + [_sspec(), _sspec()],
        out_specs=[_vspec(), _vspec()],
    )(g2p.reshape(4, TC_R, LANES), w2a, w2b, dinv,
      W2.reshape(2, 2), b2.reshape(1, 2))

    return jnp.stack([oa.reshape(NPAD)[:n], ob.reshape(NPAD)[:n]], axis=-1)
